# baseline (speedup 1.0000x reference)
import functools

import jax
import jax.numpy as jnp
from jax import lax
from jax.experimental import pallas as pl
from jax.experimental.pallas import tpu as pltpu
from jax.experimental.pallas import tpu_sc as plsc

B = 16384
D = 64
NC = 2
NS = 16
NW = NC * NS
BPW = B // NW
CHUNK = 128
NCHUNK = BPW // CHUNK
L = 16
GROUPS = BPW // L

_mesh = plsc.VectorSubcoreMesh(core_axis_name="c", subcore_axis_name="s")


@functools.partial(
    pl.kernel,
    mesh=_mesh,
    compiler_params=pltpu.CompilerParams(
        needs_layout_passes=False, use_tc_tiling_on_sc=False),
    out_type=jax.ShapeDtypeStruct((B,), jnp.float32),
    scratch_types=[
        pltpu.VMEM((NCHUNK, CHUNK), jnp.int32),
        pltpu.VMEM((NCHUNK, CHUNK), jnp.int32),
        pltpu.VMEM((BPW, D), jnp.float32),
        pltpu.VMEM((BPW, D), jnp.float32),
        pltpu.VMEM((BPW,), jnp.float32),
        pltpu.VMEM((BPW,), jnp.float32),
        pltpu.VMEM((BPW,), jnp.float32),
        pltpu.SemaphoreType.DMA,
    ],
)
def _sc_kernel(user_hbm, item_hbm, uemb_hbm, iemb_hbm, ub_hbm, ib_hbm,
               out_hbm, uidx, iidx, ue, ie, ubv, ibv, outv, sem):
    wid = lax.axis_index("s") * NC + lax.axis_index("c")
    base = wid * BPW

    pltpu.sync_copy(user_hbm.at[wid], uidx)
    pltpu.sync_copy(item_hbm.at[wid], iidx)

    copies = []
    for j in range(NCHUNK):
        sl = pl.ds(j * CHUNK, CHUNK)
        copies.append(pltpu.async_copy(uemb_hbm.at[uidx.at[j]], ue.at[sl], sem))
        copies.append(pltpu.async_copy(iemb_hbm.at[iidx.at[j]], ie.at[sl], sem))
        copies.append(pltpu.async_copy(ub_hbm.at[uidx.at[j]], ubv.at[sl], sem))
        copies.append(pltpu.async_copy(ib_hbm.at[iidx.at[j]], ibv.at[sl], sem))
    for cp in copies:
        cp.wait()

    iota = lax.iota(jnp.int32, L)

    def group_body(g, carry):
        rbase = g * L
        dotv = jnp.zeros((L,), jnp.float32)
        for r in range(L):
            row = rbase + r
            acc = ue[row, pl.ds(0, L)] * ie[row, pl.ds(0, L)]
            for c in range(1, D // L):
                acc = acc + ue[row, pl.ds(c * L, L)] * ie[row, pl.ds(c * L, L)]
            dotv = jnp.where(iota == r, jnp.sum(acc), dotv)
        outv[pl.ds(rbase, L)] = (dotv + ubv[pl.ds(rbase, L)]
                                 + ibv[pl.ds(rbase, L)])
        return carry

    lax.fori_loop(0, GROUPS, group_body, 0)

    pltpu.sync_copy(outv, out_hbm.at[pl.ds(base, BPW)])


def kernel(user, item, user_embedding, item_embedding, item_biases, user_biases):
    user_r = user.reshape(NW, NCHUNK, CHUNK)
    item_r = item.reshape(NW, NCHUNK, CHUNK)
    ub1 = user_biases.reshape(-1)
    ib1 = item_biases.reshape(-1)
    return _sc_kernel(user_r, item_r, user_embedding, item_embedding, ub1, ib1)
